# baseline (speedup 1.0000x reference)
import functools

import jax
import jax.numpy as jnp
from jax import lax
from jax.experimental import pallas as pl
from jax.experimental.pallas import tpu as pltpu
from jax.experimental.pallas import tpu_sc as plsc

_NUM_LATENTS = 1000000
_LATENT_DIM = 8
_CONTEXT_LENGTH = 200
_BATCH = 4096


@functools.lru_cache(maxsize=None)
def _build_gather():
    info = plsc.get_sparse_core_info()
    nc, ns = info.num_cores, info.num_subcores
    nw = nc * ns
    b_per_w = _BATCH // nw
    mesh = plsc.VectorSubcoreMesh(core_axis_name="c", subcore_axis_name="s")

    @functools.partial(
        pl.kernel,
        mesh=mesh,
        out_type=[
            jax.ShapeDtypeStruct((_BATCH, _LATENT_DIM), jnp.int32),
            jax.ShapeDtypeStruct((_BATCH, _CONTEXT_LENGTH), jnp.int32),
        ],
        scratch_types=[
            pltpu.VMEM((b_per_w,), jnp.int32),
            pltpu.VMEM((b_per_w, _LATENT_DIM), jnp.int32),
            pltpu.VMEM((b_per_w, _CONTEXT_LENGTH), jnp.int32),
            pltpu.SemaphoreType.DMA,
        ],
        compiler_params=pltpu.CompilerParams(use_tc_tiling_on_sc=False),
    )
    def gather_kernel(idx_hbm, lat_hbm, ids_hbm, lat_out, ids_out,
                      idx_v, lat_v, ids_v, sem):
        wid = lax.axis_index("s") * nc + lax.axis_index("c")
        base = wid * b_per_w
        pltpu.sync_copy(idx_hbm.at[pl.ds(base, b_per_w)], idx_v)
        c1 = pltpu.async_copy(lat_hbm.at[idx_v], lat_v, sem)
        c2 = pltpu.async_copy(ids_hbm.at[idx_v], ids_v, sem)
        c1.wait()
        c2.wait()
        pltpu.sync_copy(lat_v, lat_out.at[pl.ds(base, b_per_w)])
        pltpu.sync_copy(ids_v, ids_out.at[pl.ds(base, b_per_w)])

    return gather_kernel


def kernel(indices, index_to_latent, input_ids_table):
    gather = _build_gather()
    raw_latent, cond_input_ids = gather(
        indices.astype(jnp.int32), index_to_latent, input_ids_table)
    cond_ignore_mask = jnp.zeros_like(cond_input_ids, dtype=jnp.bool_)
    return (raw_latent, cond_input_ids, cond_ignore_mask)


# speedup vs baseline: 4.7309x; 4.7309x over previous
import functools

import jax
import jax.numpy as jnp
from jax import lax
from jax.experimental import pallas as pl
from jax.experimental.pallas import tpu as pltpu
from jax.experimental.pallas import tpu_sc as plsc

_NUM_LATENTS = 1000000
_LATENT_DIM = 8
_CONTEXT_LENGTH = 200
_BATCH = 4096
_REM = _CONTEXT_LENGTH - 128


@functools.lru_cache(maxsize=None)
def _build_ids_gather():
    info = plsc.get_sparse_core_info()
    nc, ns = info.num_cores, info.num_subcores
    nw = nc * ns
    b_per_w = _BATCH // nw
    mesh = plsc.VectorSubcoreMesh(core_axis_name="c", subcore_axis_name="s")

    @functools.partial(
        pl.kernel,
        mesh=mesh,
        out_type=jax.ShapeDtypeStruct((_BATCH, 256), jnp.int32),
        scratch_types=[
            pltpu.VMEM((b_per_w,), jnp.int32),
            pltpu.VMEM((b_per_w, 128), jnp.int32),
            pltpu.VMEM((32, 8, _REM), jnp.int32),
            pltpu.VMEM((b_per_w, 128), jnp.int32),
            pltpu.SemaphoreType.DMA,
            pltpu.SemaphoreType.DMA,
        ],
        compiler_params=pltpu.CompilerParams(needs_layout_passes=False),
    )
    def ids_kernel(idx_hbm, ids_hbm, ids_out, idx_v, a_v, b_v, out_v, sem_a,
                   sem_b):
        wid = lax.axis_index("s") * nc + lax.axis_index("c")
        base = wid * b_per_w
        lane = lax.broadcasted_iota(jnp.int32, (16,), 0)
        pltpu.sync_copy(idx_hbm.at[pl.ds(base, b_per_w)], idx_v)
        ca = pltpu.async_copy(ids_hbm.at[idx_v, pl.ds(0, 128)], a_v, sem_a)
        b2 = b_v.reshape(32 * 8, _REM)
        for j0 in range(0, b_per_w, 32):
            copies = []
            for jj in range(32):
                j = j0 + jj
                if j % 16 == 0:
                    x16 = idx_v[pl.ds(j, 16)]
                r = jnp.max(jnp.where(lane == (j % 16), x16, 0))
                r8 = pl.multiple_of(r - lax.rem(r, 8), 8)
                copies.append(
                    pltpu.async_copy(
                        ids_hbm.at[pl.ds(r8, 8), pl.ds(128, _REM)],
                        b_v.at[jj], sem_b,
                    )
                )
            for c in copies:
                c.wait()
            for jj in range(32):
                j = j0 + jj
                if j % 16 == 0:
                    x16 = idx_v[pl.ds(j, 16)]
                s = jnp.max(jnp.where(lane == (j % 16), x16, 0))
                row = jj * 8 + lax.rem(s, 8)
                for c in (0, 16, 32, 48, 56):
                    out_v[j, pl.ds(c, 16)] = b2[row, pl.ds(c, 16)]
        ca.wait()
        pltpu.sync_copy(a_v, ids_out.at[pl.ds(base, b_per_w), pl.ds(0, 128)])
        pltpu.sync_copy(
            out_v, ids_out.at[pl.ds(base, b_per_w), pl.ds(128, 128)]
        )

    return ids_kernel


@functools.lru_cache(maxsize=None)
def _build_latent_gather():
    info = plsc.get_sparse_core_info()
    nc, ns = info.num_cores, info.num_subcores
    nw = nc * ns
    b_per_w = _BATCH // nw
    mesh = plsc.VectorSubcoreMesh(core_axis_name="c", subcore_axis_name="s")

    @functools.partial(
        pl.kernel,
        mesh=mesh,
        out_type=jax.ShapeDtypeStruct((_BATCH, _LATENT_DIM), jnp.int32),
        scratch_types=[
            pltpu.VMEM((b_per_w,), jnp.int32),
            pltpu.VMEM((b_per_w, _LATENT_DIM), jnp.int32),
            pltpu.SemaphoreType.DMA,
        ],
        compiler_params=pltpu.CompilerParams(use_tc_tiling_on_sc=False),
    )
    def latent_kernel(idx_hbm, lat_hbm, lat_out, idx_v, lat_v, sem):
        wid = lax.axis_index("s") * nc + lax.axis_index("c")
        base = wid * b_per_w
        pltpu.sync_copy(idx_hbm.at[pl.ds(base, b_per_w)], idx_v)
        pltpu.async_copy(lat_hbm.at[idx_v], lat_v, sem).wait()
        pltpu.sync_copy(lat_v, lat_out.at[pl.ds(base, b_per_w)])

    return latent_kernel


def kernel(indices, index_to_latent, input_ids_table):
    idx = indices.astype(jnp.int32)
    ids_padded = _build_ids_gather()(idx, input_ids_table)
    cond_input_ids = ids_padded[:, :_CONTEXT_LENGTH]
    raw_latent = _build_latent_gather()(idx, index_to_latent)
    cond_ignore_mask = jnp.zeros_like(cond_input_ids, dtype=jnp.bool_)
    return (raw_latent, cond_input_ids, cond_ignore_mask)


# speedup vs baseline: 6.4474x; 1.3628x over previous
import functools

import jax
import jax.numpy as jnp
from jax import lax
from jax.experimental import pallas as pl
from jax.experimental.pallas import tpu as pltpu
from jax.experimental.pallas import tpu_sc as plsc

_NUM_LATENTS = 1000000
_LATENT_DIM = 8
_CONTEXT_LENGTH = 200
_BATCH = 4096
_REM = _CONTEXT_LENGTH - 128


@functools.lru_cache(maxsize=None)
def _build_ids_gather():
    info = plsc.get_sparse_core_info()
    nc, ns = info.num_cores, info.num_subcores
    nw = nc * ns
    b_per_w = _BATCH // nw
    mesh = plsc.VectorSubcoreMesh(core_axis_name="c", subcore_axis_name="s")

    @functools.partial(
        pl.kernel,
        mesh=mesh,
        out_type=jax.ShapeDtypeStruct((_CONTEXT_LENGTH, _BATCH), jnp.int32),
        scratch_types=[
            pltpu.VMEM((b_per_w,), jnp.int32),
            pltpu.VMEM((b_per_w, 128), jnp.int32),
            pltpu.VMEM((32, 8, _REM), jnp.int32),
            pltpu.VMEM((_CONTEXT_LENGTH, b_per_w), jnp.int32),
            pltpu.SemaphoreType.DMA,
            pltpu.SemaphoreType.DMA,
        ],
        compiler_params=pltpu.CompilerParams(needs_layout_passes=False),
    )
    def ids_kernel(idx_hbm, ids_hbm, out_hbm, idx_v, a_v, b_v, ty_v, sem_a,
                   sem_b):
        wid = lax.axis_index("s") * nc + lax.axis_index("c")
        base = wid * b_per_w
        lane = lax.broadcasted_iota(jnp.int32, (16,), 0)
        pltpu.sync_copy(idx_hbm.at[pl.ds(base, b_per_w)], idx_v)
        ca = pltpu.async_copy(ids_hbm.at[idx_v, pl.ds(0, 128)], a_v, sem_a)
        b2 = b_v.reshape(32 * 8, _REM)
        for j0 in range(0, b_per_w, 32):
            copies = []
            for jj in range(32):
                j = j0 + jj
                if j % 16 == 0:
                    x16 = idx_v[pl.ds(j, 16)]
                r = jnp.max(jnp.where(lane == (j % 16), x16, 0))
                r8 = pl.multiple_of(r - lax.rem(r, 8), 8)
                copies.append(
                    pltpu.async_copy(
                        ids_hbm.at[pl.ds(r8, 8), pl.ds(128, _REM)],
                        b_v.at[jj], sem_b,
                    )
                )
            for c in copies:
                c.wait()
            for jj in range(32):
                j = j0 + jj
                if j % 16 == 0:
                    x16 = idx_v[pl.ds(j, 16)]
                s = jnp.max(jnp.where(lane == (j % 16), x16, 0))
                row = jj * 8 + lax.rem(s, 8)
                jvec = jnp.full((16,), j, jnp.int32)
                for c in (0, 16, 32, 48, 56):
                    piece = b2[row, pl.ds(c, 16)]
                    plsc.store_scatter(ty_v, [128 + c + lane, jvec], piece)
        ca.wait()
        def _tr(c, _):
            cvec = jnp.full((16,), c, jnp.int32)
            for k in range(b_per_w // 16):
                vals = plsc.load_gather(a_v, [k * 16 + lane, cvec])
                ty_v[c, pl.ds(k * 16, 16)] = vals
            return ()

        lax.fori_loop(0, 128, _tr, (), unroll=False)
        pltpu.sync_copy(ty_v, out_hbm.at[:, pl.ds(base, b_per_w)])

    return ids_kernel


@functools.lru_cache(maxsize=None)
def _build_latent_gather():
    info = plsc.get_sparse_core_info()
    nc, ns = info.num_cores, info.num_subcores
    nw = nc * ns
    b_per_w = _BATCH // nw
    mesh = plsc.VectorSubcoreMesh(core_axis_name="c", subcore_axis_name="s")

    @functools.partial(
        pl.kernel,
        mesh=mesh,
        out_type=jax.ShapeDtypeStruct((_LATENT_DIM, _BATCH), jnp.int32),
        scratch_types=[
            pltpu.VMEM((b_per_w,), jnp.int32),
            pltpu.VMEM((32, _LATENT_DIM, 128), jnp.int32),
            pltpu.VMEM((_LATENT_DIM, b_per_w), jnp.int32),
            pltpu.SemaphoreType.DMA,
        ],
        compiler_params=pltpu.CompilerParams(needs_layout_passes=False),
    )
    def latent_kernel(idx_hbm, latt_hbm, out_hbm, idx_v, tb_v, tyl_v, sem):
        wid = lax.axis_index("s") * nc + lax.axis_index("c")
        base = wid * b_per_w
        lane = lax.broadcasted_iota(jnp.int32, (16,), 0)
        emask = lane < _LATENT_DIM
        pltpu.sync_copy(idx_hbm.at[pl.ds(base, b_per_w)], idx_v)
        for j0 in range(0, b_per_w, 32):
            copies = []
            for jj in range(32):
                j = j0 + jj
                if j % 16 == 0:
                    x16 = idx_v[pl.ds(j, 16)]
                r = jnp.max(jnp.where(lane == (j % 16), x16, 0))
                st = pl.multiple_of(r - lax.rem(r, 128), 128)
                copies.append(
                    pltpu.async_copy(
                        latt_hbm.at[:, pl.ds(st, 128)], tb_v.at[jj], sem
                    )
                )
            for c in copies:
                c.wait()
            for jj in range(32):
                j = j0 + jj
                if j % 16 == 0:
                    x16 = idx_v[pl.ds(j, 16)]
                r = jnp.max(jnp.where(lane == (j % 16), x16, 0))
                sp = lax.rem(r, 128)
                vals = plsc.load_gather(
                    tb_v.at[jj], [lane, jnp.full((16,), sp, jnp.int32)],
                    mask=emask,
                )
                plsc.store_scatter(
                    tyl_v, [lane, jnp.full((16,), j, jnp.int32)], vals,
                    mask=emask,
                )
        pltpu.sync_copy(tyl_v, out_hbm.at[:, pl.ds(base, b_per_w)])

    return latent_kernel


def kernel(indices, index_to_latent, input_ids_table):
    idx = indices.astype(jnp.int32)
    ids_t = _build_ids_gather()(idx, input_ids_table)
    cond_input_ids = jnp.transpose(ids_t)
    lat_t = _build_latent_gather()(idx, jnp.transpose(index_to_latent))
    raw_latent = jnp.transpose(lat_t)
    cond_ignore_mask = jnp.zeros_like(cond_input_ids, dtype=jnp.bool_)
    return (raw_latent, cond_input_ids, cond_ignore_mask)


# speedup vs baseline: 24.8443x; 3.8534x over previous
import functools

import jax
import jax.numpy as jnp
from jax import lax
from jax.experimental import pallas as pl
from jax.experimental.pallas import tpu as pltpu
from jax.experimental.pallas import tpu_sc as plsc

_NUM_LATENTS = 1000000
_LATENT_DIM = 8
_CONTEXT_LENGTH = 200
_BATCH = 4096
_REM = _CONTEXT_LENGTH - 128


@functools.lru_cache(maxsize=None)
def _build_ids_gather():
    info = plsc.get_sparse_core_info()
    nc, ns = info.num_cores, info.num_subcores
    nw = nc * ns
    b_per_w = _BATCH // nw
    mesh = plsc.VectorSubcoreMesh(core_axis_name="c", subcore_axis_name="s")

    nring = 3

    @functools.partial(
        pl.kernel,
        mesh=mesh,
        out_type=jax.ShapeDtypeStruct((_CONTEXT_LENGTH, _BATCH), jnp.int32),
        scratch_types=[
            pltpu.VMEM((b_per_w,), jnp.int32),
            pltpu.VMEM((nring, _CONTEXT_LENGTH, 128), jnp.int32),
            pltpu.VMEM((_CONTEXT_LENGTH, b_per_w), jnp.int32),
            [pltpu.SemaphoreType.DMA] * nring,
        ],
        compiler_params=pltpu.CompilerParams(needs_layout_passes=False),
    )
    def ids_kernel(idx_hbm, ids_hbm, out_hbm, idx_v, slab_v, ty_v, sems):
        wid = lax.axis_index("s") * nc + lax.axis_index("c")
        base = wid * b_per_w
        lane = lax.broadcasted_iota(jnp.int32, (16,), 0)
        tailmask = lane < (_CONTEXT_LENGTH - 192)
        pltpu.sync_copy(idx_hbm.at[pl.ds(base, b_per_w)], idx_v)

        def _fire(j):
            if j % 16 == 0:
                _fire.x16 = idx_v[pl.ds(j, 16)]
            r = jnp.max(jnp.where(lane == (j % 16), _fire.x16, 0))
            st = pl.multiple_of(r - lax.rem(r, 128), 128)
            return pltpu.async_copy(
                ids_hbm.at[:, pl.ds(st, 128)],
                slab_v.at[j % nring],
                sems[j % nring],
            )

        copies = [_fire(j) for j in range(nring)]
        for j in range(b_per_w):
            copies[j % nring].wait()
            if j % 16 == 0:
                x16 = idx_v[pl.ds(j, 16)]
            r = jnp.max(jnp.where(lane == (j % 16), x16, 0))
            spvec = jnp.broadcast_to(lax.rem(r, 128), (16,))
            jvec = jnp.full((16,), j, jnp.int32)
            slab = slab_v.at[j % nring]
            for c in range(0, _CONTEXT_LENGTH, 16):
                m = tailmask if c + 16 > _CONTEXT_LENGTH else None
                piece = plsc.load_gather(slab, [c + lane, spvec], mask=m)
                plsc.store_scatter(ty_v, [c + lane, jvec], piece, mask=m)
            if j + nring < b_per_w:
                copies[j % nring] = _fire(j + nring)
        pltpu.sync_copy(ty_v, out_hbm.at[:, pl.ds(base, b_per_w)])

    return ids_kernel


@functools.lru_cache(maxsize=None)
def _build_latent_gather():
    info = plsc.get_sparse_core_info()
    nc, ns = info.num_cores, info.num_subcores
    nw = nc * ns
    b_per_w = _BATCH // nw
    mesh = plsc.VectorSubcoreMesh(core_axis_name="c", subcore_axis_name="s")

    @functools.partial(
        pl.kernel,
        mesh=mesh,
        out_type=jax.ShapeDtypeStruct((_LATENT_DIM, _BATCH), jnp.int32),
        scratch_types=[
            pltpu.VMEM((b_per_w,), jnp.int32),
            pltpu.VMEM((32, _LATENT_DIM, 128), jnp.int32),
            pltpu.VMEM((_LATENT_DIM, b_per_w), jnp.int32),
            pltpu.SemaphoreType.DMA,
        ],
        compiler_params=pltpu.CompilerParams(needs_layout_passes=False),
    )
    def latent_kernel(idx_hbm, latt_hbm, out_hbm, idx_v, tb_v, tyl_v, sem):
        wid = lax.axis_index("s") * nc + lax.axis_index("c")
        base = wid * b_per_w
        lane = lax.broadcasted_iota(jnp.int32, (16,), 0)
        emask = lane < _LATENT_DIM
        pltpu.sync_copy(idx_hbm.at[pl.ds(base, b_per_w)], idx_v)
        for j0 in range(0, b_per_w, 32):
            copies = []
            for jj in range(32):
                j = j0 + jj
                if j % 16 == 0:
                    x16 = idx_v[pl.ds(j, 16)]
                r = jnp.max(jnp.where(lane == (j % 16), x16, 0))
                st = pl.multiple_of(r - lax.rem(r, 128), 128)
                copies.append(
                    pltpu.async_copy(
                        latt_hbm.at[:, pl.ds(st, 128)], tb_v.at[jj], sem
                    )
                )
            for c in copies:
                c.wait()
            for jj in range(32):
                j = j0 + jj
                if j % 16 == 0:
                    x16 = idx_v[pl.ds(j, 16)]
                r = jnp.max(jnp.where(lane == (j % 16), x16, 0))
                sp = lax.rem(r, 128)
                vals = plsc.load_gather(
                    tb_v.at[jj], [lane, jnp.full((16,), sp, jnp.int32)],
                    mask=emask,
                )
                plsc.store_scatter(
                    tyl_v, [lane, jnp.full((16,), j, jnp.int32)], vals,
                    mask=emask,
                )
        pltpu.sync_copy(tyl_v, out_hbm.at[:, pl.ds(base, b_per_w)])

    return latent_kernel


def kernel(indices, index_to_latent, input_ids_table):
    idx = indices.astype(jnp.int32)
    ids_t = _build_ids_gather()(idx, jnp.transpose(input_ids_table))
    cond_input_ids = jnp.transpose(ids_t)
    lat_t = _build_latent_gather()(idx, jnp.transpose(index_to_latent))
    raw_latent = jnp.transpose(lat_t)
    cond_ignore_mask = jnp.zeros_like(cond_input_ids, dtype=jnp.bool_)
    return (raw_latent, cond_input_ids, cond_ignore_mask)


# speedup vs baseline: 27.1208x; 1.0916x over previous
import functools

import jax
import jax.numpy as jnp
from jax import lax
from jax.experimental import pallas as pl
from jax.experimental.pallas import tpu as pltpu
from jax.experimental.pallas import tpu_sc as plsc

_NUM_LATENTS = 1000000
_LATENT_DIM = 8
_CONTEXT_LENGTH = 200
_BATCH = 4096


@functools.lru_cache(maxsize=None)
def _build_gather():
    info = plsc.get_sparse_core_info()
    nc, ns = info.num_cores, info.num_subcores
    nw = nc * ns
    b_per_w = _BATCH // nw
    mesh = plsc.VectorSubcoreMesh(core_axis_name="c", subcore_axis_name="s")
    nring = 3

    @functools.partial(
        pl.kernel,
        mesh=mesh,
        out_type=[
            jax.ShapeDtypeStruct((_CONTEXT_LENGTH, _BATCH), jnp.int32),
            jax.ShapeDtypeStruct((_LATENT_DIM, _BATCH), jnp.int32),
        ],
        scratch_types=[
            pltpu.VMEM((b_per_w,), jnp.int32),
            pltpu.VMEM((nring, _CONTEXT_LENGTH, 128), jnp.int32),
            pltpu.VMEM((nring, _LATENT_DIM, 128), jnp.int32),
            pltpu.VMEM((_CONTEXT_LENGTH, b_per_w), jnp.int32),
            pltpu.VMEM((_LATENT_DIM, b_per_w), jnp.int32),
            [pltpu.SemaphoreType.DMA] * nring,
            [pltpu.SemaphoreType.DMA] * nring,
        ],
        compiler_params=pltpu.CompilerParams(needs_layout_passes=False),
    )
    def gather_kernel(idx_hbm, ids_hbm, lat_hbm, ids_out, lat_out,
                      idx_v, slab_v, lslab_v, ty_v, tyl_v, sems, lsems):
        wid = lax.axis_index("s") * nc + lax.axis_index("c")
        base = wid * b_per_w
        lane = lax.broadcasted_iota(jnp.int32, (16,), 0)
        tailmask = lane < (_CONTEXT_LENGTH - 192)
        lmask = lane < _LATENT_DIM
        pltpu.sync_copy(idx_hbm.at[pl.ds(base, b_per_w)], idx_v)

        def _r_of(j, x16):
            return jnp.max(jnp.where(lane == (j % 16), x16, 0))

        def _fire(j, x16):
            r = _r_of(j, x16)
            st = pl.multiple_of(r - lax.rem(r, 128), 128)
            return (
                pltpu.async_copy(
                    ids_hbm.at[:, pl.ds(st, 128)],
                    slab_v.at[j % nring], sems[j % nring],
                ),
                pltpu.async_copy(
                    lat_hbm.at[:, pl.ds(st, 128)],
                    lslab_v.at[j % nring], lsems[j % nring],
                ),
            )

        chunks = [idx_v[pl.ds(k * 16, 16)] for k in range(b_per_w // 16)]
        copies = [_fire(j, chunks[0]) for j in range(nring)]
        for j in range(b_per_w):
            ca, cl = copies[j % nring]
            ca.wait()
            cl.wait()
            r = _r_of(j, chunks[j // 16])
            spvec = jnp.broadcast_to(lax.rem(r, 128), (16,))
            jvec = jnp.full((16,), j, jnp.int32)
            slab = slab_v.at[j % nring]
            for c in range(0, _CONTEXT_LENGTH, 16):
                m = tailmask if c + 16 > _CONTEXT_LENGTH else None
                piece = plsc.load_gather(slab, [c + lane, spvec], mask=m)
                plsc.store_scatter(ty_v, [c + lane, jvec], piece, mask=m)
            lpiece = plsc.load_gather(
                lslab_v.at[j % nring], [lane, spvec], mask=lmask
            )
            plsc.store_scatter(tyl_v, [lane, jvec], lpiece, mask=lmask)
            if j + nring < b_per_w:
                copies[j % nring] = _fire(j + nring, chunks[(j + nring) // 16])
        pltpu.sync_copy(ty_v, ids_out.at[:, pl.ds(base, b_per_w)])
        pltpu.sync_copy(tyl_v, lat_out.at[:, pl.ds(base, b_per_w)])

    return gather_kernel


def kernel(indices, index_to_latent, input_ids_table):
    idx = indices.astype(jnp.int32)
    ids_t, lat_t = _build_gather()(
        idx, jnp.transpose(input_ids_table), jnp.transpose(index_to_latent)
    )
    cond_input_ids = jnp.transpose(ids_t)
    raw_latent = jnp.transpose(lat_t)
    cond_ignore_mask = jnp.zeros_like(cond_input_ids, dtype=jnp.bool_)
    return (raw_latent, cond_input_ids, cond_ignore_mask)
